# Initial kernel scaffold; baseline (speedup 1.0000x reference)
import jax
import jax.numpy as jnp
from jax.experimental import pallas as pl


def kernel(kmer_indices, embedding_weight):
    raise NotImplementedError("write your pallas kernel here")


# speedup vs baseline: 3.1784x; 3.1784x over previous
import functools

import jax
import jax.numpy as jnp
from jax import lax
from jax.experimental import pallas as pl
from jax.experimental.pallas import tpu as pltpu
from jax.experimental.pallas import tpu_sc as plsc

_D = 64
_B_TOTAL = 4096 * 200

_info = plsc.get_sparse_core_info()
_NC = _info.num_cores
_NS = _info.num_subcores
_NW = _NC * _NS
_B_PER_W = _B_TOTAL // _NW
_CHUNK = 128
_N_CHUNKS = _B_PER_W // _CHUNK

_mesh = plsc.VectorSubcoreMesh(core_axis_name="c", subcore_axis_name="s")


@functools.partial(
    pl.kernel,
    mesh=_mesh,
    out_type=jax.ShapeDtypeStruct((_B_TOTAL, _D), jnp.float32),
    scratch_types=[
        pltpu.VMEM((_CHUNK,), jnp.int32),
        pltpu.VMEM((_CHUNK, _D), jnp.float32),
        pltpu.SemaphoreType.DMA,
    ],
    compiler_params=pltpu.CompilerParams(use_tc_tiling_on_sc=False),
)
def _gather_rows(idx_hbm, table_hbm, out_hbm, idx_v, rows_v, sem):
    wid = lax.axis_index("s") * _NC + lax.axis_index("c")
    base = wid * _B_PER_W

    def body(i, _):
        off = base + i * _CHUNK
        pltpu.sync_copy(idx_hbm.at[pl.ds(off, _CHUNK)], idx_v)
        pltpu.async_copy(table_hbm.at[idx_v], rows_v, sem).wait()
        pltpu.sync_copy(rows_v, out_hbm.at[pl.ds(off, _CHUNK)])
        return 0

    lax.fori_loop(0, _N_CHUNKS, body, 0)


def kernel(kmer_indices, embedding_weight):
    flat_idx = kmer_indices.reshape(-1).astype(jnp.int32)
    out = _gather_rows(flat_idx, embedding_weight)
    return out.reshape(kmer_indices.shape + (embedding_weight.shape[-1],))


# speedup vs baseline: 4.2475x; 1.3364x over previous
import functools

import jax
import jax.numpy as jnp
from jax import lax
from jax.experimental import pallas as pl
from jax.experimental.pallas import tpu as pltpu
from jax.experimental.pallas import tpu_sc as plsc

_D = 64
_B_TOTAL = 4096 * 200

_info = plsc.get_sparse_core_info()
_NC = _info.num_cores
_NS = _info.num_subcores
_NW = _NC * _NS
_B_PER_W = _B_TOTAL // _NW
_CHUNK = 256
_NBUF = 4
_N_CHUNKS = _B_PER_W // _CHUNK
_N_GROUPS = _N_CHUNKS // _NBUF

_mesh = plsc.VectorSubcoreMesh(core_axis_name="c", subcore_axis_name="s")


@functools.partial(
    pl.kernel,
    mesh=_mesh,
    out_type=jax.ShapeDtypeStruct((_B_TOTAL, _D), jnp.float32),
    scratch_types=[
        pltpu.VMEM((_B_PER_W,), jnp.int32),
        pltpu.VMEM((_NBUF, _CHUNK, _D), jnp.float32),
        pltpu.SemaphoreType.DMA((_NBUF,)),
        pltpu.SemaphoreType.DMA((_NBUF,)),
    ],
    compiler_params=pltpu.CompilerParams(use_tc_tiling_on_sc=False),
)
def _gather_rows(idx_hbm, table_hbm, out_hbm, idx_v, rows_v, gsem, ssem):
    wid = lax.axis_index("s") * _NC + lax.axis_index("c")
    base = wid * _B_PER_W

    pltpu.sync_copy(idx_hbm.at[pl.ds(base, _B_PER_W)], idx_v)

    def group(g, _):
        goff = g * (_NBUF * _CHUNK)
        for b in range(_NBUF):
            off = goff + b * _CHUNK

            @pl.when(g > 0)
            def _():
                pltpu.make_async_copy(
                    rows_v.at[b], out_hbm.at[pl.ds(base, _CHUNK)], ssem.at[b]
                ).wait()

            pltpu.async_copy(
                table_hbm.at[idx_v.at[pl.ds(off, _CHUNK)]],
                rows_v.at[b],
                gsem.at[b],
            )
        for b in range(_NBUF):
            off = goff + b * _CHUNK
            pltpu.make_async_copy(
                table_hbm.at[idx_v.at[pl.ds(off, _CHUNK)]],
                rows_v.at[b],
                gsem.at[b],
            ).wait()
            pltpu.async_copy(
                rows_v.at[b], out_hbm.at[pl.ds(base + off, _CHUNK)], ssem.at[b]
            )
        return 0

    lax.fori_loop(0, _N_GROUPS, group, 0)

    for b in range(_NBUF):
        pltpu.make_async_copy(
            rows_v.at[b], out_hbm.at[pl.ds(base, _CHUNK)], ssem.at[b]
        ).wait()


def kernel(kmer_indices, embedding_weight):
    flat_idx = kmer_indices.reshape(-1).astype(jnp.int32)
    out = _gather_rows(flat_idx, embedding_weight)
    return out.reshape(kmer_indices.shape + (embedding_weight.shape[-1],))
